# speedup vs baseline: 5.7057x; 5.7057x over previous
import functools

import jax
import jax.numpy as jnp
from jax import lax
from jax.experimental import pallas as pl
from jax.experimental.pallas import tpu as pltpu
from jax.experimental.pallas import tpu_sc as plsc

BATCH = 1024
SEQ = 512
D = 128
NTOK = BATCH * SEQ

_info = plsc.get_sparse_core_info()
NC, NS, L = _info.num_cores, _info.num_subcores, _info.num_lanes
NW = NC * NS
PER_W = NTOK // NW
CHUNK = 128
N_CHUNKS = PER_W // CHUNK


def _comb_kernel(pe_ref, seg_ref, out_ref):
    out_ref[...] = pe_ref[...][None, :, :] + seg_ref[...][:, None, :]


def _build_comb(pe, segment_table):
    return pl.pallas_call(
        _comb_kernel,
        out_shape=jax.ShapeDtypeStruct((2, SEQ, D), jnp.float32),
    )(pe, segment_table)


def _make_sc_kernel():
    mesh = plsc.VectorSubcoreMesh(core_axis_name="c", subcore_axis_name="s")

    @functools.partial(
        pl.kernel,
        mesh=mesh,
        out_type=jax.ShapeDtypeStruct((NTOK, D), jnp.float32),
        scratch_types=[
            pltpu.VMEM((CHUNK,), jnp.int32),
            pltpu.VMEM((CHUNK,), jnp.int32),
            pltpu.VMEM((CHUNK,), jnp.int32),
            pltpu.VMEM((CHUNK, D), jnp.float32),
            pltpu.VMEM((CHUNK, D), jnp.float32),
            pltpu.SemaphoreType.DMA,
            pltpu.SemaphoreType.DMA,
        ],
    )
    def k(tok_hbm, ids_hbm, seg_hbm, comb_hbm, out_hbm,
          idx_v, segv, cidx_v, tok_rows, comb_rows, sem_a, sem_b):
        wid = lax.axis_index("s") * NC + lax.axis_index("c")
        wbase = wid * PER_W
        iota = lax.iota(jnp.int32, L)

        def chunk_body(c, _):
            base = wbase + c * CHUNK
            pos_base = lax.rem(base, SEQ)
            pltpu.sync_copy(ids_hbm.at[pl.ds(base, CHUNK)], idx_v)
            pltpu.sync_copy(seg_hbm.at[pl.ds(base, CHUNK)], segv)

            def cidx_body(i, _):
                sl = pl.ds(i * L, L)
                cidx_v[sl] = segv[sl] * SEQ + (pos_base + i * L + iota)
                return 0

            lax.fori_loop(0, CHUNK // L, cidx_body, 0)

            cp_a = pltpu.async_copy(tok_hbm.at[idx_v], tok_rows, sem_a)
            cp_b = pltpu.async_copy(comb_hbm.at[cidx_v], comb_rows, sem_b)
            cp_a.wait()
            cp_b.wait()

            def add_body(i, _):
                for j in range(D // L):
                    sl = pl.ds(j * L, L)
                    tok_rows[i, sl] = tok_rows[i, sl] + comb_rows[i, sl]
                return 0

            lax.fori_loop(0, CHUNK, add_body, 0)
            pltpu.sync_copy(tok_rows, out_hbm.at[pl.ds(base, CHUNK)])
            return 0

        lax.fori_loop(0, N_CHUNKS, chunk_body, 0)

    return k


_sc_kernel = _make_sc_kernel()


def kernel(input_ids, segment_ids, token_table, segment_table, pe):
    comb = _build_comb(pe, segment_table).reshape(2 * SEQ, D)
    ids = input_ids.reshape(NTOK).astype(jnp.int32)
    segs = segment_ids.reshape(NTOK).astype(jnp.int32)
    out = _sc_kernel(token_table, ids, segs, comb)
    return out.reshape(BATCH, SEQ, D)


# speedup vs baseline: 5.8084x; 1.0180x over previous
import functools

import jax
import jax.numpy as jnp
from jax import lax
from jax.experimental import pallas as pl
from jax.experimental.pallas import tpu as pltpu
from jax.experimental.pallas import tpu_sc as plsc

BATCH = 1024
SEQ = 512
D = 128
NTOK = BATCH * SEQ

_info = plsc.get_sparse_core_info()
NC, NS, L = _info.num_cores, _info.num_subcores, _info.num_lanes
NW = NC * NS
PER_W = NTOK // NW
CHUNK = 128
N_CHUNKS = PER_W // CHUNK


def _comb_kernel(pe_ref, seg_ref, out_ref):
    out_ref[...] = pe_ref[...][None, :, :] + seg_ref[...][:, None, :]


def _build_comb(pe, segment_table):
    return pl.pallas_call(
        _comb_kernel,
        out_shape=jax.ShapeDtypeStruct((2, SEQ, D), jnp.float32),
    )(pe, segment_table)


def _make_sc_kernel():
    mesh = plsc.VectorSubcoreMesh(core_axis_name="c", subcore_axis_name="s")

    @functools.partial(
        pl.kernel,
        mesh=mesh,
        out_type=jax.ShapeDtypeStruct((NTOK, D), jnp.float32),
        scratch_types=[
            pltpu.VMEM((CHUNK,), jnp.int32),
            pltpu.VMEM((CHUNK,), jnp.int32),
            pltpu.VMEM((CHUNK,), jnp.int32),
            pltpu.VMEM((CHUNK, D), jnp.float32),
            pltpu.VMEM((CHUNK, D), jnp.float32),
            pltpu.SemaphoreType.DMA,
            pltpu.SemaphoreType.DMA,
        ],
    )
    def k(tok_hbm, ids_hbm, seg_hbm, comb_hbm, out_hbm,
          idx_v, segv, cidx_v, tok_rows, comb_rows, sem_a, sem_b):
        wid = lax.axis_index("s") * NC + lax.axis_index("c")
        wbase = wid * PER_W
        iota = lax.iota(jnp.int32, L)

        def chunk_body(c, _):
            base = wbase + c * CHUNK
            pos_base = lax.rem(base, SEQ)
            pltpu.sync_copy(ids_hbm.at[pl.ds(base, CHUNK)], idx_v)
            pltpu.sync_copy(seg_hbm.at[pl.ds(base, CHUNK)], segv)

            def cidx_body(i, _):
                sl = pl.ds(i * L, L)
                cidx_v[sl] = segv[sl] * SEQ + (pos_base + i * L + iota)
                return 0

            lax.fori_loop(0, CHUNK // L, cidx_body, 0)

            cp_b = pltpu.async_copy(comb_hbm.at[cidx_v], tok_rows, sem_b)
            cp_b.wait()
            cp_a = pltpu.async_copy(tok_hbm.at[idx_v], tok_rows, sem_a, add=True)
            cp_a.wait()
            pltpu.sync_copy(tok_rows, out_hbm.at[pl.ds(base, CHUNK)])
            return 0

        lax.fori_loop(0, N_CHUNKS, chunk_body, 0)

    return k


_sc_kernel = _make_sc_kernel()


def kernel(input_ids, segment_ids, token_table, segment_table, pe):
    comb = _build_comb(pe, segment_table).reshape(2 * SEQ, D)
    ids = input_ids.reshape(NTOK).astype(jnp.int32)
    segs = segment_ids.reshape(NTOK).astype(jnp.int32)
    out = _sc_kernel(token_table, ids, segs, comb)
    return out.reshape(BATCH, SEQ, D)


# speedup vs baseline: 9.6832x; 1.6671x over previous
import functools

import jax
import jax.numpy as jnp
from jax import lax
from jax.experimental import pallas as pl
from jax.experimental.pallas import tpu as pltpu
from jax.experimental.pallas import tpu_sc as plsc

BATCH = 1024
SEQ = 512
D = 128
NTOK = BATCH * SEQ

_info = plsc.get_sparse_core_info()
NC, NS, L = _info.num_cores, _info.num_subcores, _info.num_lanes
NW = NC * NS
PER_W = NTOK // NW
CHUNK = 128
NBUF = 4
N_CHUNKS = PER_W // CHUNK
N_GROUPS = N_CHUNKS // NBUF


def _comb_kernel(pe_ref, seg_ref, out_ref):
    out_ref[...] = pe_ref[...][None, :, :] + seg_ref[...][:, None, :]


def _build_comb(pe, segment_table):
    return pl.pallas_call(
        _comb_kernel,
        out_shape=jax.ShapeDtypeStruct((2, SEQ, D), jnp.float32),
    )(pe, segment_table)


def _make_sc_kernel():
    mesh = plsc.VectorSubcoreMesh(core_axis_name="c", subcore_axis_name="s")

    @functools.partial(
        pl.kernel,
        mesh=mesh,
        out_type=jax.ShapeDtypeStruct((NTOK, D), jnp.float32),
        scratch_types=[
            pltpu.VMEM((NBUF, CHUNK), jnp.int32),
            pltpu.VMEM((NBUF, CHUNK), jnp.int32),
            pltpu.VMEM((NBUF, CHUNK), jnp.int32),
            pltpu.VMEM((NBUF, CHUNK, D), jnp.float32),
            pltpu.SemaphoreType.DMA((NBUF,)),
            pltpu.SemaphoreType.DMA((NBUF,)),
            pltpu.SemaphoreType.DMA((NBUF,)),
            pltpu.SemaphoreType.DMA((NBUF,)),
            pltpu.SemaphoreType.DMA((NBUF,)),
        ],
    )
    def k(tok_hbm, ids_hbm, seg_hbm, comb_hbm, out_hbm,
          idx_v, segv, cidx_v, rows, sem_i, sem_s, sem_c, sem_t, sem_o):
        wid = lax.axis_index("s") * NC + lax.axis_index("c")
        wbase = wid * PER_W
        iota = lax.iota(jnp.int32, L)

        def fetch_ids(c, b):
            base = wbase + c * CHUNK
            pltpu.async_copy(ids_hbm.at[pl.ds(base, CHUNK)], idx_v.at[b],
                             sem_i.at[b])
            pltpu.async_copy(seg_hbm.at[pl.ds(base, CHUNK)], segv.at[b],
                             sem_s.at[b])

        def wait_ids(b):
            pltpu.make_async_copy(ids_hbm.at[pl.ds(0, CHUNK)], idx_v.at[b],
                                  sem_i.at[b]).wait()
            pltpu.make_async_copy(seg_hbm.at[pl.ds(0, CHUNK)], segv.at[b],
                                  sem_s.at[b]).wait()

        def wait_out(b):
            pltpu.make_async_copy(rows.at[b], out_hbm.at[pl.ds(0, CHUNK)],
                                  sem_o.at[b]).wait()

        def stage_comb(c, b):
            wait_ids(b)
            pos_base = lax.rem(wbase + c * CHUNK, SEQ)
            for i in range(CHUNK // L):
                sl = pl.ds(i * L, L)
                cidx_v[b, sl] = segv[b, sl] * SEQ + (pos_base + i * L + iota)
            pltpu.async_copy(comb_hbm.at[cidx_v.at[b]], rows.at[b],
                             sem_c.at[b])

        def stage_tok(b):
            pltpu.make_async_copy(comb_hbm.at[cidx_v.at[b]], rows.at[b],
                                  sem_c.at[b]).wait()
            pltpu.async_copy(tok_hbm.at[idx_v.at[b]], rows.at[b],
                             sem_t.at[b], add=True)

        def stage_out(c, b):
            base = wbase + c * CHUNK
            pltpu.make_async_copy(tok_hbm.at[idx_v.at[b]], rows.at[b],
                                  sem_t.at[b]).wait()
            pltpu.async_copy(rows.at[b], out_hbm.at[pl.ds(base, CHUNK)],
                             sem_o.at[b])

        for b in range(NBUF):
            fetch_ids(b, b)
        for b in range(NBUF):
            stage_comb(b, b)
        for b in range(NBUF):
            stage_tok(b)
        for b in range(NBUF):
            stage_out(b, b)
            fetch_ids(NBUF + b, b)

        def group_body(g, _):
            c0 = g * NBUF
            for b in range(NBUF):
                wait_out(b)
                stage_comb(c0 + b, b)
            for b in range(NBUF):
                stage_tok(b)
            for b in range(NBUF):
                stage_out(c0 + b, b)
                cn = lax.min(c0 + NBUF + b, jnp.int32(N_CHUNKS - 1))
                fetch_ids(cn, b)
            return 0

        lax.fori_loop(1, N_GROUPS, group_body, 0)

        for b in range(NBUF):
            wait_out(b)
            wait_ids(b)

    return k


_sc_kernel = _make_sc_kernel()


def kernel(input_ids, segment_ids, token_table, segment_table, pe):
    comb = _build_comb(pe, segment_table).reshape(2 * SEQ, D)
    ids = input_ids.reshape(NTOK).astype(jnp.int32)
    segs = segment_ids.reshape(NTOK).astype(jnp.int32)
    out = _sc_kernel(token_table, ids, segs, comb)
    return out.reshape(BATCH, SEQ, D)


# speedup vs baseline: 17.1310x; 1.7691x over previous
import functools

import jax
import jax.numpy as jnp
from jax import lax
from jax.experimental import pallas as pl
from jax.experimental.pallas import tpu as pltpu
from jax.experimental.pallas import tpu_sc as plsc

BATCH = 1024
SEQ = 512
D = 128
NTOK = BATCH * SEQ

_info = plsc.get_sparse_core_info()
NC, NS, L = _info.num_cores, _info.num_subcores, _info.num_lanes
NW = NC * NS
PER_W = NTOK // NW
CHUNK = 128
NBUF = 4
N_CHUNKS = PER_W // CHUNK
N_GROUPS = N_CHUNKS // NBUF


def _comb_kernel(pe_ref, seg_ref, out_ref):
    out_ref[...] = pe_ref[...][None, :, :] + seg_ref[...][:, None, :]


def _build_comb(pe, segment_table):
    return pl.pallas_call(
        _comb_kernel,
        out_shape=jax.ShapeDtypeStruct((2, SEQ, D), jnp.float32),
    )(pe, segment_table)


def _make_sc_kernel():
    mesh = plsc.VectorSubcoreMesh(core_axis_name="c", subcore_axis_name="s")

    @functools.partial(
        pl.kernel,
        mesh=mesh,
        out_type=jax.ShapeDtypeStruct((NTOK, D), jnp.float32),
        scratch_types=[
            pltpu.VMEM((NBUF, CHUNK), jnp.int32),
            pltpu.VMEM((NBUF, CHUNK), jnp.int32),
            pltpu.VMEM((NBUF, CHUNK), jnp.int32),
            pltpu.VMEM((NBUF, CHUNK, D), jnp.float32),
            pltpu.VMEM_SHARED((2 * SEQ, D), jnp.float32),
            pltpu.SemaphoreType.DMA((NBUF,)),
            pltpu.SemaphoreType.DMA((NBUF,)),
            pltpu.SemaphoreType.DMA((NBUF,)),
            pltpu.SemaphoreType.DMA((NBUF,)),
            pltpu.SemaphoreType.DMA((NBUF,)),
        ],
    )
    def k(tok_hbm, ids_hbm, seg_hbm, comb_hbm, out_hbm,
          idx_v, segv, cidx_v, rows, comb_sh,
          sem_i, sem_s, sem_c, sem_t, sem_o):
        sid = lax.axis_index("s")
        wid = sid * NC + lax.axis_index("c")
        wbase = wid * PER_W
        iota = lax.iota(jnp.int32, L)

        @pl.when(sid == 0)
        def _():
            pltpu.sync_copy(comb_hbm, comb_sh)

        plsc.subcore_barrier()

        def fetch_ids(c, b):
            base = wbase + c * CHUNK
            pltpu.async_copy(ids_hbm.at[pl.ds(base, CHUNK)], idx_v.at[b],
                             sem_i.at[b])
            pltpu.async_copy(seg_hbm.at[pl.ds(base, CHUNK)], segv.at[b],
                             sem_s.at[b])

        def wait_ids(b):
            pltpu.make_async_copy(ids_hbm.at[pl.ds(0, CHUNK)], idx_v.at[b],
                                  sem_i.at[b]).wait()
            pltpu.make_async_copy(seg_hbm.at[pl.ds(0, CHUNK)], segv.at[b],
                                  sem_s.at[b]).wait()

        def wait_out(b):
            pltpu.make_async_copy(rows.at[b], out_hbm.at[pl.ds(0, CHUNK)],
                                  sem_o.at[b]).wait()

        def stage_comb(c, b):
            wait_ids(b)
            pos_base = lax.rem(wbase + c * CHUNK, SEQ)
            for i in range(CHUNK // L):
                sl = pl.ds(i * L, L)
                cidx_v[b, sl] = segv[b, sl] * SEQ + (pos_base + i * L + iota)
            pltpu.async_copy(comb_sh.at[cidx_v.at[b]], rows.at[b],
                             sem_c.at[b])

        def stage_tok(b):
            pltpu.make_async_copy(comb_sh.at[cidx_v.at[b]], rows.at[b],
                                  sem_c.at[b]).wait()
            pltpu.async_copy(tok_hbm.at[idx_v.at[b]], rows.at[b],
                             sem_t.at[b], add=True)

        def stage_out(c, b):
            base = wbase + c * CHUNK
            pltpu.make_async_copy(tok_hbm.at[idx_v.at[b]], rows.at[b],
                                  sem_t.at[b]).wait()
            pltpu.async_copy(rows.at[b], out_hbm.at[pl.ds(base, CHUNK)],
                             sem_o.at[b])

        for b in range(NBUF):
            fetch_ids(b, b)
        for b in range(NBUF):
            stage_comb(b, b)
        for b in range(NBUF):
            stage_tok(b)
        for b in range(NBUF):
            stage_out(b, b)
            fetch_ids(NBUF + b, b)

        def group_body(g, _):
            c0 = g * NBUF
            for b in range(NBUF):
                wait_out(b)
                stage_comb(c0 + b, b)
            for b in range(NBUF):
                stage_tok(b)
            for b in range(NBUF):
                stage_out(c0 + b, b)
                cn = lax.min(c0 + NBUF + b, jnp.int32(N_CHUNKS - 1))
                fetch_ids(cn, b)
            return 0

        lax.fori_loop(1, N_GROUPS, group_body, 0)

        for b in range(NBUF):
            wait_out(b)
            wait_ids(b)

    return k


_sc_kernel = _make_sc_kernel()


def kernel(input_ids, segment_ids, token_table, segment_table, pe):
    comb = _build_comb(pe, segment_table).reshape(2 * SEQ, D)
    ids = input_ids.reshape(NTOK).astype(jnp.int32)
    segs = segment_ids.reshape(NTOK).astype(jnp.int32)
    out = _sc_kernel(token_table, ids, segs, comb)
    return out.reshape(BATCH, SEQ, D)
